# baseline (speedup 1.0000x reference)
import functools

import jax
import jax.numpy as jnp
from jax import lax
from jax.experimental import pallas as pl
from jax.experimental.pallas import tpu as pltpu
from jax.experimental.pallas import tpu_sc as plsc

_MIN_LEVEL = 2
_MAX_LEVEL = 6
_OUT = 7
_OFFSET = 0.5
_NW = 32
_F = 256
_NS = 2 * _OUT
_NPTS = _NS * _NS


def _prep(boxes, sizes):
    batch_size, num_boxes = boxes.shape[0], boxes.shape[1]
    heights = [s for s in sizes]
    widths = [s for s in sizes]
    offsets = [0]
    for h, w in zip(heights[:-1], widths[:-1]):
        offsets.append(offsets[-1] + h * w)
    batch_dim_size = offsets[-1] + heights[-1] * widths[-1]
    level_dim_offsets = jnp.asarray(offsets, jnp.int32)
    height_dim_sizes = jnp.asarray(widths, jnp.int32)
    max_h, max_w = float(heights[0]), float(widths[0])

    box_width = boxes[:, :, 3] - boxes[:, :, 1]
    box_height = boxes[:, :, 2] - boxes[:, :, 0]
    areas_sqrt = jnp.sqrt(box_height.astype(jnp.float32) * box_width.astype(jnp.float32))
    levels = (jnp.floor(jnp.log(areas_sqrt / 224.0) / jnp.log(2.0)) + 4.0).astype(jnp.int32)
    levels = jnp.minimum(_MAX_LEVEL, jnp.maximum(levels, _MIN_LEVEL))
    scale_to_level = jnp.power(2.0, levels.astype(boxes.dtype))
    boxes = boxes / scale_to_level[..., None]
    box_width = box_width / scale_to_level
    box_height = box_height / scale_to_level
    boxes = jnp.concatenate(
        [boxes[:, :, 0:2], box_height[..., None], box_width[..., None]], axis=-1)

    levels = levels - _MIN_LEVEL
    level_strides = jnp.power(2.0, levels.astype(jnp.float32))
    boundary_h = (max_h / level_strides - 1.0).astype(boxes.dtype)
    boundary_w = (max_w / level_strides - 1.0).astype(boxes.dtype)

    i = jnp.arange(_OUT, dtype=boxes.dtype)
    box_grid_x = boxes[:, :, 1:2] + (i + _OFFSET)[None, None, :] * boxes[:, :, 3:4] / _OUT
    box_grid_y = boxes[:, :, 0:1] + (i + _OFFSET)[None, None, :] * boxes[:, :, 2:3] / _OUT
    box_grid_y0 = jnp.maximum(0.0, jnp.floor(box_grid_y))
    box_grid_x0 = jnp.maximum(0.0, jnp.floor(box_grid_x))
    box_grid_x0 = jnp.minimum(box_grid_x0, boundary_w[..., None])
    box_grid_x1 = jnp.minimum(box_grid_x0 + 1, boundary_w[..., None])
    box_grid_y0 = jnp.minimum(box_grid_y0, boundary_h[..., None])
    box_grid_y1 = jnp.minimum(box_grid_y0 + 1, boundary_h[..., None])
    ly = box_grid_y - box_grid_y0
    lx = box_grid_x - box_grid_x0
    hy = 1.0 - ly
    hx = 1.0 - lx

    x_idx = jnp.stack([box_grid_x0, box_grid_x1], axis=-1).reshape(
        batch_size, num_boxes, _NS).astype(jnp.int32)
    y_idx = jnp.stack([box_grid_y0, box_grid_y1], axis=-1).reshape(
        batch_size, num_boxes, _NS).astype(jnp.int32)

    batch_off = (jnp.arange(batch_size, dtype=jnp.int32) * batch_dim_size).reshape(
        batch_size, 1, 1, 1)
    level_off = level_dim_offsets[levels].reshape(batch_size, num_boxes, 1, 1)
    y_off = (y_idx * height_dim_sizes[levels][..., None]).reshape(
        batch_size, num_boxes, _NS, 1)
    x_off = x_idx.reshape(batch_size, num_boxes, 1, _NS)
    idx = (batch_off + level_off + y_off + x_off).reshape(
        batch_size * num_boxes, 2, _NPTS // 2)

    wx = jnp.stack([hx, lx], axis=-1).reshape(batch_size, num_boxes, _NS)
    wy = jnp.stack([hy, ly], axis=-1).reshape(batch_size, num_boxes, _NS)
    w = jnp.concatenate([wx, wy], axis=-1).reshape(batch_size * num_boxes, 28)
    w = jnp.broadcast_to(w[:, :, None], (batch_size * num_boxes, 28, 16))
    return idx, w


def _roi_sc_body(feat_hbm, idx_hbm, w_hbm, out_hbm,
                 idx_v, w_v, rows_v, out_v,
                 semf00, semf01, semw0, semf10, semf11, semw1, semo):
    c = lax.axis_index("c")
    s = lax.axis_index("s")
    wid = s * 2 + c
    bpw = idx_hbm.shape[0] // _NW
    base = wid * bpw
    pltpu.sync_copy(idx_hbm.at[pl.ds(base, bpw)], idx_v)

    sems = [(semf00, semf01, semw0), (semf10, semf11, semw1)]

    def gather_box(i, b):
        s0, s1, sw = sems[b]
        pltpu.async_copy(feat_hbm.at[idx_v.at[i, 0]], rows_v.at[b, 0], s0)
        pltpu.async_copy(feat_hbm.at[idx_v.at[i, 1]], rows_v.at[b, 1], s1)
        pltpu.async_copy(w_hbm.at[base + i], w_v.at[b], sw)

    def wait_box(i, b):
        s0, s1, sw = sems[b]
        pltpu.make_async_copy(feat_hbm.at[idx_v.at[i, 0]], rows_v.at[b, 0], s0).wait()
        pltpu.make_async_copy(feat_hbm.at[idx_v.at[i, 1]], rows_v.at[b, 1], s1).wait()
        pltpu.make_async_copy(w_hbm.at[base + i], w_v.at[b], sw).wait()

    gather_box(0, 0)
    gather_box(1, 1)

    def pair_body(g, carry):
        i0 = 2 * g
        for b in range(2):
            i = i0 + b
            wait_box(i, b)

            @pl.when(i > 0)
            def _drain_out():
                pltpu.make_async_copy(out_v, out_hbm.at[base], semo).wait()

            def ci_body(ci, carry_ci, b=b):
                for u in range(4):
                    sl = pl.ds(ci * 64 + u * 16, 16)
                    for p in range(_OUT):
                        wy0 = w_v[b, 14 + 2 * p]
                        wy1 = w_v[b, 15 + 2 * p]
                        y_t, y_b = 2 * p, 2 * p + 1
                        h_t, h_b = int(y_t >= _OUT), int(y_b >= _OUT)
                        rb_t = y_t * _NS - h_t * (_NPTS // 2)
                        rb_b = y_b * _NS - h_b * (_NPTS // 2)
                        for q in range(_OUT):
                            wx0 = w_v[b, 2 * q]
                            wx1 = w_v[b, 2 * q + 1]
                            rt = rb_t + 2 * q
                            rb = rb_b + 2 * q
                            top = (rows_v[b, h_t, rt, sl] * wx0
                                   + rows_v[b, h_t, rt + 1, sl] * wx1)
                            bot = (rows_v[b, h_b, rb, sl] * wx0
                                   + rows_v[b, h_b, rb + 1, sl] * wx1)
                            out_v[p * _OUT + q, sl] = top * wy0 + bot * wy1
                return carry_ci

            lax.fori_loop(0, _F // 64, ci_body, 0)

            pltpu.async_copy(out_v, out_hbm.at[base + i], semo)

            @pl.when(i + 2 < bpw)
            def _prefetch(i=i, b=b):
                gather_box(i + 2, b)
        return carry

    lax.fori_loop(0, bpw // 2, pair_body, 0)
    pltpu.make_async_copy(out_v, out_hbm.at[base], semo).wait()


@jax.jit
def kernel(features_2, features_3, features_4, features_5, features_6, boxes):
    feats = [features_2, features_3, features_4, features_5, features_6]
    batch_size = feats[0].shape[0]
    num_boxes = boxes.shape[1]
    sizes = [f.shape[1] for f in feats]
    feats_r2 = jnp.concatenate(
        [f.reshape(batch_size, -1, _F) for f in feats], axis=1).reshape(-1, _F)

    idx, w = _prep(boxes, sizes)
    nb_total = batch_size * num_boxes

    mesh = plsc.VectorSubcoreMesh(core_axis_name="c", subcore_axis_name="s")
    out = pl.kernel(
        _roi_sc_body,
        out_type=jax.ShapeDtypeStruct((nb_total, _OUT * _OUT, _F), jnp.float32),
        mesh=mesh,
        scratch_types=[
            pltpu.VMEM((nb_total // _NW, 2, _NPTS // 2), jnp.int32),
            pltpu.VMEM((2, 28, 16), jnp.float32),
            pltpu.VMEM((2, 2, _NPTS // 2, _F), jnp.float32),
            pltpu.VMEM((_OUT * _OUT, _F), jnp.float32),
            pltpu.SemaphoreType.DMA,
            pltpu.SemaphoreType.DMA,
            pltpu.SemaphoreType.DMA,
            pltpu.SemaphoreType.DMA,
            pltpu.SemaphoreType.DMA,
            pltpu.SemaphoreType.DMA,
            pltpu.SemaphoreType.DMA,
        ],
        compiler_params=pltpu.CompilerParams(use_tc_tiling_on_sc=False),
    )(feats_r2, idx, w)
    return out.reshape(batch_size, num_boxes, _OUT, _OUT, _F)


# speedup vs baseline: 1.8830x; 1.8830x over previous
import functools

import jax
import jax.numpy as jnp
from jax import lax
from jax.experimental import pallas as pl
from jax.experimental.pallas import tpu as pltpu
from jax.experimental.pallas import tpu_sc as plsc

_MIN_LEVEL = 2
_MAX_LEVEL = 6
_OUT = 7
_OFFSET = 0.5
_NW = 32
_F = 256
_NS = 2 * _OUT
_NPTS = _NS * _NS


def _prep(boxes, sizes):
    batch_size, num_boxes = boxes.shape[0], boxes.shape[1]
    heights = [s for s in sizes]
    widths = [s for s in sizes]
    offsets = [0]
    for h, w in zip(heights[:-1], widths[:-1]):
        offsets.append(offsets[-1] + h * w)
    batch_dim_size = offsets[-1] + heights[-1] * widths[-1]
    level_dim_offsets = jnp.asarray(offsets, jnp.int32)
    height_dim_sizes = jnp.asarray(widths, jnp.int32)
    max_h, max_w = float(heights[0]), float(widths[0])

    box_width = boxes[:, :, 3] - boxes[:, :, 1]
    box_height = boxes[:, :, 2] - boxes[:, :, 0]
    areas_sqrt = jnp.sqrt(box_height.astype(jnp.float32) * box_width.astype(jnp.float32))
    levels = (jnp.floor(jnp.log(areas_sqrt / 224.0) / jnp.log(2.0)) + 4.0).astype(jnp.int32)
    levels = jnp.minimum(_MAX_LEVEL, jnp.maximum(levels, _MIN_LEVEL))
    scale_to_level = jnp.power(2.0, levels.astype(boxes.dtype))
    boxes = boxes / scale_to_level[..., None]
    box_width = box_width / scale_to_level
    box_height = box_height / scale_to_level
    boxes = jnp.concatenate(
        [boxes[:, :, 0:2], box_height[..., None], box_width[..., None]], axis=-1)

    levels = levels - _MIN_LEVEL
    level_strides = jnp.power(2.0, levels.astype(jnp.float32))
    boundary_h = (max_h / level_strides - 1.0).astype(boxes.dtype)
    boundary_w = (max_w / level_strides - 1.0).astype(boxes.dtype)

    i = jnp.arange(_OUT, dtype=boxes.dtype)
    box_grid_x = boxes[:, :, 1:2] + (i + _OFFSET)[None, None, :] * boxes[:, :, 3:4] / _OUT
    box_grid_y = boxes[:, :, 0:1] + (i + _OFFSET)[None, None, :] * boxes[:, :, 2:3] / _OUT
    box_grid_y0 = jnp.maximum(0.0, jnp.floor(box_grid_y))
    box_grid_x0 = jnp.maximum(0.0, jnp.floor(box_grid_x))
    box_grid_x0 = jnp.minimum(box_grid_x0, boundary_w[..., None])
    box_grid_x1 = jnp.minimum(box_grid_x0 + 1, boundary_w[..., None])
    box_grid_y0 = jnp.minimum(box_grid_y0, boundary_h[..., None])
    box_grid_y1 = jnp.minimum(box_grid_y0 + 1, boundary_h[..., None])
    ly = box_grid_y - box_grid_y0
    lx = box_grid_x - box_grid_x0
    hy = 1.0 - ly
    hx = 1.0 - lx

    x_idx = jnp.stack([box_grid_x0, box_grid_x1], axis=-1).reshape(
        batch_size, num_boxes, _NS).astype(jnp.int32)
    y_idx = jnp.stack([box_grid_y0, box_grid_y1], axis=-1).reshape(
        batch_size, num_boxes, _NS).astype(jnp.int32)

    batch_off = (jnp.arange(batch_size, dtype=jnp.int32) * batch_dim_size).reshape(
        batch_size, 1, 1, 1)
    level_off = level_dim_offsets[levels].reshape(batch_size, num_boxes, 1, 1)
    y_off = (y_idx * height_dim_sizes[levels][..., None]).reshape(
        batch_size, num_boxes, _NS, 1)
    x_off = x_idx.reshape(batch_size, num_boxes, 1, _NS)
    idx = (batch_off + level_off + y_off + x_off).reshape(
        batch_size * num_boxes, 2, _NPTS // 2)

    wx = jnp.stack([hx, lx], axis=-1).reshape(batch_size, num_boxes, _NS)
    wy = jnp.stack([hy, ly], axis=-1).reshape(batch_size, num_boxes, _NS)
    w = jnp.concatenate([wx, wy], axis=-1).reshape(batch_size * num_boxes, 28)
    w = jnp.broadcast_to(w[:, :, None], (batch_size * num_boxes, 28, 16))
    return idx, w


def _roi_sc_body(feat_hbm, idx_hbm, w_hbm, out_hbm,
                 idx_v, w_v, rows_v, out_v,
                 semf00, semf01, semw0, semf10, semf11, semw1, semo):
    c = lax.axis_index("c")
    s = lax.axis_index("s")
    wid = s * 2 + c
    bpw = idx_hbm.shape[0] // _NW
    base = wid * bpw
    pltpu.sync_copy(idx_hbm.at[pl.ds(base, bpw)], idx_v)

    sems = [(semf00, semf01, semw0), (semf10, semf11, semw1)]

    def gather_box(i, b):
        s0, s1, sw = sems[b]
        pltpu.async_copy(feat_hbm.at[idx_v.at[i, 0]], rows_v.at[b, 0], s0)
        pltpu.async_copy(feat_hbm.at[idx_v.at[i, 1]], rows_v.at[b, 1], s1)
        pltpu.async_copy(w_hbm.at[base + i], w_v.at[b], sw)

    def wait_box(i, b):
        s0, s1, sw = sems[b]
        pltpu.make_async_copy(feat_hbm.at[idx_v.at[i, 0]], rows_v.at[b, 0], s0).wait()
        pltpu.make_async_copy(feat_hbm.at[idx_v.at[i, 1]], rows_v.at[b, 1], s1).wait()
        pltpu.make_async_copy(w_hbm.at[base + i], w_v.at[b], sw).wait()

    gather_box(0, 0)
    gather_box(1, 1)

    def pair_body(g, carry):
        i0 = 2 * g
        for b in range(2):
            i = i0 + b
            wait_box(i, b)

            @pl.when(i > 0)
            def _drain_out():
                pltpu.make_async_copy(out_v, out_hbm.at[base], semo).wait()

            def ci_body(ci, carry_ci, b=b):
                sl = pl.ds(ci * 16, 16)
                wxv = [w_v[b, k] for k in range(_NS)]
                for p in range(_OUT):
                    wy0 = w_v[b, 14 + 2 * p]
                    wy1 = w_v[b, 15 + 2 * p]
                    y_t, y_b = 2 * p, 2 * p + 1
                    h_t, h_b = int(y_t >= _OUT), int(y_b >= _OUT)
                    rb_t = y_t * _NS - h_t * (_NPTS // 2)
                    rb_b = y_b * _NS - h_b * (_NPTS // 2)
                    for q in range(_OUT):
                        wx0 = wxv[2 * q]
                        wx1 = wxv[2 * q + 1]
                        rt = rb_t + 2 * q
                        rb = rb_b + 2 * q
                        top = (rows_v[b, h_t, rt, sl] * wx0
                               + rows_v[b, h_t, rt + 1, sl] * wx1)
                        bot = (rows_v[b, h_b, rb, sl] * wx0
                               + rows_v[b, h_b, rb + 1, sl] * wx1)
                        out_v[p * _OUT + q, sl] = top * wy0 + bot * wy1
                return carry_ci

            lax.fori_loop(0, _F // 16, ci_body, 0)

            pltpu.async_copy(out_v, out_hbm.at[base + i], semo)

            @pl.when(i + 2 < bpw)
            def _prefetch(i=i, b=b):
                gather_box(i + 2, b)
        return carry

    lax.fori_loop(0, bpw // 2, pair_body, 0)
    pltpu.make_async_copy(out_v, out_hbm.at[base], semo).wait()


@jax.jit
def kernel(features_2, features_3, features_4, features_5, features_6, boxes):
    feats = [features_2, features_3, features_4, features_5, features_6]
    batch_size = feats[0].shape[0]
    num_boxes = boxes.shape[1]
    sizes = [f.shape[1] for f in feats]
    feats_r2 = jnp.concatenate(
        [f.reshape(batch_size, -1, _F) for f in feats], axis=1).reshape(-1, _F)

    idx, w = _prep(boxes, sizes)
    nb_total = batch_size * num_boxes

    mesh = plsc.VectorSubcoreMesh(core_axis_name="c", subcore_axis_name="s")
    out = pl.kernel(
        _roi_sc_body,
        out_type=jax.ShapeDtypeStruct((nb_total, _OUT * _OUT, _F), jnp.float32),
        mesh=mesh,
        scratch_types=[
            pltpu.VMEM((nb_total // _NW, 2, _NPTS // 2), jnp.int32),
            pltpu.VMEM((2, 28, 16), jnp.float32),
            pltpu.VMEM((2, 2, _NPTS // 2, _F), jnp.float32),
            pltpu.VMEM((_OUT * _OUT, _F), jnp.float32),
            pltpu.SemaphoreType.DMA,
            pltpu.SemaphoreType.DMA,
            pltpu.SemaphoreType.DMA,
            pltpu.SemaphoreType.DMA,
            pltpu.SemaphoreType.DMA,
            pltpu.SemaphoreType.DMA,
            pltpu.SemaphoreType.DMA,
        ],
        compiler_params=pltpu.CompilerParams(use_tc_tiling_on_sc=False),
    )(feats_r2, idx, w)
    return out.reshape(batch_size, num_boxes, _OUT, _OUT, _F)


# speedup vs baseline: 1.8859x; 1.0015x over previous
import functools

import jax
import jax.numpy as jnp
from jax import lax
from jax.experimental import pallas as pl
from jax.experimental.pallas import tpu as pltpu
from jax.experimental.pallas import tpu_sc as plsc

_MIN_LEVEL = 2
_MAX_LEVEL = 6
_OUT = 7
_OFFSET = 0.5
_NW = 32
_F = 256
_NS = 2 * _OUT
_NPTS = _NS * _NS


def _prep(boxes, sizes):
    batch_size, num_boxes = boxes.shape[0], boxes.shape[1]
    heights = [s for s in sizes]
    widths = [s for s in sizes]
    offsets = [0]
    for h, w in zip(heights[:-1], widths[:-1]):
        offsets.append(offsets[-1] + h * w)
    batch_dim_size = offsets[-1] + heights[-1] * widths[-1]
    level_dim_offsets = jnp.asarray(offsets, jnp.int32)
    height_dim_sizes = jnp.asarray(widths, jnp.int32)
    max_h, max_w = float(heights[0]), float(widths[0])

    box_width = boxes[:, :, 3] - boxes[:, :, 1]
    box_height = boxes[:, :, 2] - boxes[:, :, 0]
    areas_sqrt = jnp.sqrt(box_height.astype(jnp.float32) * box_width.astype(jnp.float32))
    levels = (jnp.floor(jnp.log(areas_sqrt / 224.0) / jnp.log(2.0)) + 4.0).astype(jnp.int32)
    levels = jnp.minimum(_MAX_LEVEL, jnp.maximum(levels, _MIN_LEVEL))
    scale_to_level = jnp.power(2.0, levels.astype(boxes.dtype))
    boxes = boxes / scale_to_level[..., None]
    box_width = box_width / scale_to_level
    box_height = box_height / scale_to_level
    boxes = jnp.concatenate(
        [boxes[:, :, 0:2], box_height[..., None], box_width[..., None]], axis=-1)

    levels = levels - _MIN_LEVEL
    level_strides = jnp.power(2.0, levels.astype(jnp.float32))
    boundary_h = (max_h / level_strides - 1.0).astype(boxes.dtype)
    boundary_w = (max_w / level_strides - 1.0).astype(boxes.dtype)

    i = jnp.arange(_OUT, dtype=boxes.dtype)
    box_grid_x = boxes[:, :, 1:2] + (i + _OFFSET)[None, None, :] * boxes[:, :, 3:4] / _OUT
    box_grid_y = boxes[:, :, 0:1] + (i + _OFFSET)[None, None, :] * boxes[:, :, 2:3] / _OUT
    box_grid_y0 = jnp.maximum(0.0, jnp.floor(box_grid_y))
    box_grid_x0 = jnp.maximum(0.0, jnp.floor(box_grid_x))
    box_grid_x0 = jnp.minimum(box_grid_x0, boundary_w[..., None])
    box_grid_x1 = jnp.minimum(box_grid_x0 + 1, boundary_w[..., None])
    box_grid_y0 = jnp.minimum(box_grid_y0, boundary_h[..., None])
    box_grid_y1 = jnp.minimum(box_grid_y0 + 1, boundary_h[..., None])
    ly = box_grid_y - box_grid_y0
    lx = box_grid_x - box_grid_x0
    hy = 1.0 - ly
    hx = 1.0 - lx

    x_idx = jnp.stack([box_grid_x0, box_grid_x1], axis=-1).reshape(
        batch_size, num_boxes, _NS).astype(jnp.int32)
    y_idx = jnp.stack([box_grid_y0, box_grid_y1], axis=-1).reshape(
        batch_size, num_boxes, _NS).astype(jnp.int32)

    batch_off = (jnp.arange(batch_size, dtype=jnp.int32) * batch_dim_size).reshape(
        batch_size, 1, 1, 1)
    level_off = level_dim_offsets[levels].reshape(batch_size, num_boxes, 1, 1)
    y_off = (y_idx * height_dim_sizes[levels][..., None]).reshape(
        batch_size, num_boxes, _NS, 1)
    x_off = x_idx.reshape(batch_size, num_boxes, 1, _NS)
    idx = (batch_off + level_off + y_off + x_off).reshape(
        batch_size * num_boxes, 4, _NPTS // 4)

    wx = jnp.stack([hx, lx], axis=-1).reshape(batch_size, num_boxes, _NS)
    wy = jnp.stack([hy, ly], axis=-1).reshape(batch_size, num_boxes, _NS)
    w = jnp.concatenate([wx, wy], axis=-1).reshape(batch_size * num_boxes, 28)
    w = jnp.broadcast_to(w[:, :, None], (batch_size * num_boxes, 28, 16))
    return idx, w


def _roi_sc_body(feat_hbm, idx_hbm, w_hbm, out_hbm,
                 idx_v, w_v, rows_v, out_v,
                 semf00, semf01, semw0, semf10, semf11, semw1, semo):
    c = lax.axis_index("c")
    s = lax.axis_index("s")
    wid = s * 2 + c
    bpw = idx_hbm.shape[0] // _NW
    base = wid * bpw
    pltpu.sync_copy(idx_hbm.at[pl.ds(base, bpw)], idx_v)

    sems = [(semf00, semf01, semw0), (semf10, semf11, semw1)]

    def gather_box(i, b):
        s0, s1, sw = sems[b]
        for k in range(4):
            pltpu.async_copy(feat_hbm.at[idx_v.at[i, k]], rows_v.at[b, k],
                             s0 if k < 2 else s1)
        pltpu.async_copy(w_hbm.at[base + i], w_v.at[b], sw)

    def wait_box(i, b):
        s0, s1, sw = sems[b]
        for k in range(4):
            pltpu.make_async_copy(feat_hbm.at[idx_v.at[i, k]], rows_v.at[b, k],
                                  s0 if k < 2 else s1).wait()
        pltpu.make_async_copy(w_hbm.at[base + i], w_v.at[b], sw).wait()

    gather_box(0, 0)
    gather_box(1, 1)

    def pair_body(g, carry):
        i0 = 2 * g
        for b in range(2):
            i = i0 + b
            wait_box(i, b)

            @pl.when(i > 0)
            def _drain_out():
                pltpu.make_async_copy(out_v, out_hbm.at[base], semo).wait()

            def ci_body(ci, carry_ci, b=b):
                sl = pl.ds(ci * 16, 16)
                wxv = [w_v[b, k] for k in range(_NS)]

                def rref(r, b=b, sl=sl):
                    return rows_v[b, r // 49, r % 49, sl]

                for p in range(_OUT):
                    wy0 = w_v[b, 14 + 2 * p]
                    wy1 = w_v[b, 15 + 2 * p]
                    rt0 = (2 * p) * _NS
                    rb0 = (2 * p + 1) * _NS
                    for q in range(_OUT):
                        wx0 = wxv[2 * q]
                        wx1 = wxv[2 * q + 1]
                        rt = rt0 + 2 * q
                        rb = rb0 + 2 * q
                        top = rref(rt) * wx0 + rref(rt + 1) * wx1
                        bot = rref(rb) * wx0 + rref(rb + 1) * wx1
                        out_v[p * _OUT + q, sl] = top * wy0 + bot * wy1
                return carry_ci

            lax.fori_loop(0, _F // 16, ci_body, 0)

            pltpu.async_copy(out_v, out_hbm.at[base + i], semo)

            @pl.when(i + 2 < bpw)
            def _prefetch(i=i, b=b):
                gather_box(i + 2, b)
        return carry

    lax.fori_loop(0, bpw // 2, pair_body, 0)
    pltpu.make_async_copy(out_v, out_hbm.at[base], semo).wait()


@jax.jit
def kernel(features_2, features_3, features_4, features_5, features_6, boxes):
    feats = [features_2, features_3, features_4, features_5, features_6]
    batch_size = feats[0].shape[0]
    num_boxes = boxes.shape[1]
    sizes = [f.shape[1] for f in feats]
    feats_r2 = jnp.concatenate(
        [f.reshape(batch_size, -1, _F) for f in feats], axis=1).reshape(-1, _F)

    idx, w = _prep(boxes, sizes)
    nb_total = batch_size * num_boxes

    mesh = plsc.VectorSubcoreMesh(core_axis_name="c", subcore_axis_name="s")
    out = pl.kernel(
        _roi_sc_body,
        out_type=jax.ShapeDtypeStruct((nb_total, _OUT * _OUT, _F), jnp.float32),
        mesh=mesh,
        scratch_types=[
            pltpu.VMEM((nb_total // _NW, 4, _NPTS // 4), jnp.int32),
            pltpu.VMEM((2, 28, 16), jnp.float32),
            pltpu.VMEM((2, 4, _NPTS // 4, _F), jnp.float32),
            pltpu.VMEM((_OUT * _OUT, _F), jnp.float32),
            pltpu.SemaphoreType.DMA,
            pltpu.SemaphoreType.DMA,
            pltpu.SemaphoreType.DMA,
            pltpu.SemaphoreType.DMA,
            pltpu.SemaphoreType.DMA,
            pltpu.SemaphoreType.DMA,
            pltpu.SemaphoreType.DMA,
        ],
        compiler_params=pltpu.CompilerParams(use_tc_tiling_on_sc=False),
    )(feats_r2, idx, w)
    return out.reshape(batch_size, num_boxes, _OUT, _OUT, _F)
